# baseline (speedup 1.0000x reference)
import functools

import jax
import jax.numpy as jnp
from jax import lax
from jax.experimental import pallas as pl
from jax.experimental.pallas import tpu as pltpu
from jax.experimental.pallas import tpu_sc as plsc

N_NODES = 10000
N_EDGES = 320000
D = 128
N_CAND = 5000

NC = 2
NS = 16
NW = NC * NS

EDGES_PER_W = N_EDGES // NW
CHUNK = 80
N_STAGES = 5
SCHUNKS = 25
N_CHUNKS = N_STAGES * SCHUNKS
EPS = EDGES_PER_W // N_STAGES

CAND_PAD = 5120
CAND_PER_TILE = CAND_PAD // NS
CAND_CHUNKS = CAND_PER_TILE // CHUNK


def _sc_kernel(constraint, cflat, vflat, wflat, cand_pad, zeros):
  mesh = plsc.VectorSubcoreMesh(
      core_axis_name="c", subcore_axis_name="s", num_cores=NC, num_subcores=NS)

  @functools.partial(
      pl.kernel,
      mesh=mesh,
      out_type=jax.ShapeDtypeStruct((NC, CAND_PAD, D), jnp.float32),
      scratch_types=[
          pltpu.VMEM((2 * 2048,), jnp.int32),
          pltpu.VMEM((4, 1, CHUNK), jnp.int32),
          pltpu.VMEM((2 * 2048,), jnp.float32),
          pltpu.VMEM((CAND_CHUNKS, CHUNK), jnp.int32),
          pltpu.VMEM((CHUNK, D), jnp.float32),
          pltpu.VMEM((CHUNK, D), jnp.float32),
          pltpu.VMEM((CHUNK, D), jnp.float32),
          pltpu.VMEM((CHUNK, D), jnp.float32),
          pltpu.VMEM_SHARED((N_NODES, D), jnp.float32),
          pltpu.SemaphoreType.DMA,
          pltpu.SemaphoreType.DMA,
          pltpu.SemaphoreType.DMA,
          pltpu.SemaphoreType.DMA,
          pltpu.SemaphoreType.DMA,
          pltpu.SemaphoreType.DMA,
          pltpu.SemaphoreType.DMA,
          pltpu.SemaphoreType.DMA,
          pltpu.SemaphoreType.DMA,
      ],
  )
  def k(constraint_hbm, c_hbm, v_hbm, w_hbm, cand_hbm, zeros_hbm, outp_hbm,
        cb, vst, wb, q2d, rows0, rows1, rows2, rows3, acc,
        isem, gsem0, gsem1, gsem2, gsem3, ssem0, ssem1, ssem2, ssem3):
    c = lax.axis_index("c")
    s = lax.axis_index("s")
    wid = c * NS + s
    ebase = wid * EDGES_PER_W

    gbufs = ((rows0, gsem0, ssem0), (rows1, gsem1, ssem1),
             (rows2, gsem2, ssem2), (rows3, gsem3, ssem3))

    def ioff(ib):
      return pl.multiple_of(ib * 2048, 8)

    def stage_copies(t, ib):
      off = pl.multiple_of(ebase + t * EPS, 8)
      pltpu.async_copy(c_hbm.at[pl.ds(off, EPS)],
                       cb.at[pl.ds(ioff(ib), EPS)], isem)
      pltpu.async_copy(w_hbm.at[pl.ds(off, EPS)],
                       wb.at[pl.ds(ioff(ib), EPS)], isem)

    def stage_waits(t, ib):
      off = pl.multiple_of(ebase + t * EPS, 8)
      pltpu.make_async_copy(c_hbm.at[pl.ds(off, EPS)],
                            cb.at[pl.ds(ioff(ib), EPS)], isem).wait()
      pltpu.make_async_copy(w_hbm.at[pl.ds(off, EPS)],
                            wb.at[pl.ds(ioff(ib), EPS)], isem).wait()

    def vcopy(kk1, slot):
      voff = pl.multiple_of(ebase + kk1 * CHUNK, 8)
      return v_hbm.at[pl.ds(voff, CHUNK)], vst.at[slot, 0]

    stage_copies(0, 0)

    @pl.when(s < NS - 1)
    def _():
      pltpu.sync_copy(zeros_hbm.at[pl.ds(s * 640, 640)],
                      acc.at[pl.ds(s * 640, 640)])

    @pl.when(s == NS - 1)
    def _():
      pltpu.sync_copy(zeros_hbm.at[pl.ds(9600, 400)],
                      acc.at[pl.ds(9600, 400)])

    stage_waits(0, 0)
    vsrc0, vdst0 = vcopy(0, 0)
    pltpu.async_copy(vsrc0, vdst0, gsem0)
    pltpu.async_copy(
        constraint_hbm.at[cb.at[pl.ds(0, CHUNK)]], rows0, gsem0)
    vsrc1, vdst1 = vcopy(1, 1)
    pltpu.async_copy(vsrc1, vdst1, gsem1)
    pltpu.async_copy(
        constraint_hbm.at[cb.at[pl.ds(CHUNK, CHUNK)]], rows1, gsem1)

    plsc.subcore_barrier()

    def chunk_body(kk, _):
      t = lax.div(kk, SCHUNKS)
      r = lax.rem(kk, SCHUNKS)
      ib = lax.rem(t, 2)

      @pl.when((r == 0) & (t + 1 < N_STAGES))
      def _():
        stage_copies(t + 1, lax.rem(t + 1, 2))

      for b in range(4):
        rbuf, gsem, ssem = gbufs[b]
        nb = (b + 2) % 4
        nbuf, ngsem, nssem = gbufs[nb]

        @pl.when(lax.rem(kk, 4) == b)
        def _(rbuf=rbuf, gsem=gsem, ssem=ssem,
              nbuf=nbuf, ngsem=ngsem, nssem=nssem):
          vsrc, vdst = vcopy(kk, lax.rem(kk, 4))
          pltpu.make_async_copy(vsrc, vdst, gsem).wait()
          coff = pl.multiple_of(ioff(ib) + r * CHUNK, 8)
          pltpu.make_async_copy(
              constraint_hbm.at[cb.at[pl.ds(coff, CHUNK)]],
              rbuf, gsem).wait()

          @pl.when(kk >= 2)
          def _():
            pltpu.make_async_copy(
                nbuf, acc.at[vst.at[0, 0]], nssem).wait()

          @pl.when(kk + 2 < N_CHUNKS)
          def _():
            kk2 = kk + 2
            t2 = lax.div(kk2, SCHUNKS)
            r2 = lax.rem(kk2, SCHUNKS)
            ib2 = lax.rem(t2, 2)

            @pl.when(r2 == 0)
            def _():
              stage_waits(t2, ib2)

            vsrc2, vdst2 = vcopy(kk2, lax.rem(kk2, 4))
            pltpu.async_copy(vsrc2, vdst2, ngsem)
            coff2 = pl.multiple_of(ioff(ib2) + r2 * CHUNK, 8)
            pltpu.async_copy(
                constraint_hbm.at[cb.at[pl.ds(coff2, CHUNK)]],
                nbuf, ngsem)

          def scale_grp(g, _):
            w16 = wb[pl.ds(ioff(ib) + r * CHUNK + g * 16, 16)]
            for l in range(16):
              wsc = w16[l]
              e = g * 16 + l
              for j in range(D // 16):
                rbuf[e, pl.ds(j * 16, 16)] = rbuf[e, pl.ds(j * 16, 16)] * wsc
            return 0

          lax.fori_loop(0, CHUNK // 16, scale_grp, 0)

          pltpu.async_copy(
              rbuf, acc.at[vst.at[lax.rem(kk, 4), 0]], ssem, add=True)

      return 0

    lax.fori_loop(0, N_CHUNKS, chunk_body, 0)

    pltpu.make_async_copy(rows3, acc.at[vst.at[0, 0]], ssem3).wait()
    pltpu.make_async_copy(rows0, acc.at[vst.at[0, 0]], ssem0).wait()
    plsc.subcore_barrier()

    pltpu.sync_copy(cand_hbm.at[s], q2d)
    for q in range(CAND_CHUNKS):
      rbuf, sem, _ = gbufs[q % 2]
      cbase = s * CAND_PER_TILE + q * CHUNK
      if q >= 2:
        pbase = s * CAND_PER_TILE + (q - 2) * CHUNK
        pltpu.make_async_copy(
            rbuf, outp_hbm.at[c, pl.ds(pbase, CHUNK)], sem).wait()
      pltpu.sync_copy(acc.at[q2d.at[q]], rbuf)
      pltpu.async_copy(rbuf, outp_hbm.at[c, pl.ds(cbase, CHUNK)], sem)
    for q in range(CAND_CHUNKS - 2, CAND_CHUNKS):
      rbuf, sem, _ = gbufs[q % 2]
      cbase = s * CAND_PER_TILE + q * CHUNK
      pltpu.make_async_copy(
          rbuf, outp_hbm.at[c, pl.ds(cbase, CHUNK)], sem).wait()

  return k(constraint, cflat, vflat, wflat, cand_pad, zeros)


def _combine(partials):
  def body(a_ref, b_ref, o_ref):
    o_ref[...] = a_ref[...] + b_ref[...]

  blk = 40
  flat = partials.reshape(NC * CAND_PAD, D)
  off = CAND_PAD // blk
  return pl.pallas_call(
      body,
      grid=(N_CAND // blk,),
      in_specs=[pl.BlockSpec((blk, D), lambda i: (i, 0)),
                pl.BlockSpec((blk, D), lambda i: (i + off, 0))],
      out_specs=pl.BlockSpec((blk, D), lambda i: (i, 0)),
      out_shape=jax.ShapeDtypeStruct((N_CAND, D), jnp.float32),
  )(flat, flat)


def kernel(constraint, variable, cv_edge_index, edge_attr, cand_mask):
  cflat = cv_edge_index[0]
  vflat = cv_edge_index[1]
  wflat = edge_attr[:, 0]
  cand_pad = jnp.concatenate(
      [cand_mask, jnp.zeros((CAND_PAD - N_CAND,), jnp.int32)]
  ).reshape(NS, CAND_CHUNKS, CHUNK)
  zeros = jnp.zeros_like(variable)
  partials = _sc_kernel(constraint, cflat, vflat, wflat, cand_pad, zeros)
  return _combine(partials)


# speedup vs baseline: 1.3370x; 1.3370x over previous
import functools

import jax
import jax.numpy as jnp
from jax import lax
from jax.experimental import pallas as pl
from jax.experimental.pallas import tpu as pltpu
from jax.experimental.pallas import tpu_sc as plsc

N_NODES = 10000
N_EDGES = 320000
D = 128
N_CAND = 5000

NC = 2
NS = 16
NW = NC * NS

EDGES_PER_W = N_EDGES // NW
CHUNK = 80
N_STAGES = 5
SCHUNKS = 25
N_CHUNKS = N_STAGES * SCHUNKS
EPS = EDGES_PER_W // N_STAGES

CAND_PAD = 5120
CAND_PER_TILE = CAND_PAD // NS
CAND_CHUNKS = CAND_PER_TILE // CHUNK


def _sc_kernel(constraint, cflat, vflat, wflat, cand_pad, zeros):
  mesh = plsc.VectorSubcoreMesh(
      core_axis_name="c", subcore_axis_name="s", num_cores=NC, num_subcores=NS)

  @functools.partial(
      pl.kernel,
      mesh=mesh,
      out_type=jax.ShapeDtypeStruct((NC, CAND_PAD, D), jnp.float32),
      scratch_types=[
          pltpu.VMEM((2 * 2048,), jnp.int32),
          pltpu.VMEM((4, 1, CHUNK), jnp.int32),
          pltpu.VMEM((2 * 2048,), jnp.float32),
          pltpu.VMEM((CAND_CHUNKS, CHUNK), jnp.int32),
          pltpu.VMEM((CHUNK, D), jnp.float32),
          pltpu.VMEM((CHUNK, D), jnp.float32),
          pltpu.VMEM((CHUNK, D), jnp.float32),
          pltpu.VMEM((CHUNK, D), jnp.float32),
          pltpu.VMEM_SHARED((N_NODES, D), jnp.float32),
          pltpu.SemaphoreType.DMA,
          pltpu.SemaphoreType.DMA,
          pltpu.SemaphoreType.DMA,
          pltpu.SemaphoreType.DMA,
          pltpu.SemaphoreType.DMA,
          pltpu.SemaphoreType.DMA,
          pltpu.SemaphoreType.DMA,
          pltpu.SemaphoreType.DMA,
          pltpu.SemaphoreType.DMA,
      ],
  )
  def k(constraint_hbm, c_hbm, v_hbm, w_hbm, cand_hbm, zeros_hbm, outp_hbm,
        cb, vst, wb, q2d, rows0, rows1, rows2, rows3, acc,
        isem, gsem0, gsem1, gsem2, gsem3, ssem0, ssem1, ssem2, ssem3):
    c = lax.axis_index("c")
    s = lax.axis_index("s")
    wid = c * NS + s
    ebase = wid * EDGES_PER_W

    gbufs = ((rows0, gsem0, ssem0), (rows1, gsem1, ssem1),
             (rows2, gsem2, ssem2), (rows3, gsem3, ssem3))

    def ioff(ib):
      return pl.multiple_of(ib * 2048, 8)

    def stage_copies(t, ib):
      off = pl.multiple_of(ebase + t * EPS, 8)
      pltpu.async_copy(c_hbm.at[pl.ds(off, EPS)],
                       cb.at[pl.ds(ioff(ib), EPS)], isem)
      pltpu.async_copy(w_hbm.at[pl.ds(off, EPS)],
                       wb.at[pl.ds(ioff(ib), EPS)], isem)

    def stage_waits(t, ib):
      off = pl.multiple_of(ebase + t * EPS, 8)
      pltpu.make_async_copy(c_hbm.at[pl.ds(off, EPS)],
                            cb.at[pl.ds(ioff(ib), EPS)], isem).wait()
      pltpu.make_async_copy(w_hbm.at[pl.ds(off, EPS)],
                            wb.at[pl.ds(ioff(ib), EPS)], isem).wait()

    def vcopy(kk1, slot):
      voff = pl.multiple_of(ebase + kk1 * CHUNK, 8)
      return v_hbm.at[pl.ds(voff, CHUNK)], vst.at[slot, 0]

    stage_copies(0, 0)

    @pl.when(s < NS - 1)
    def _():
      pltpu.sync_copy(zeros_hbm.at[pl.ds(s * 640, 640)],
                      acc.at[pl.ds(s * 640, 640)])

    @pl.when(s == NS - 1)
    def _():
      pltpu.sync_copy(zeros_hbm.at[pl.ds(9600, 400)],
                      acc.at[pl.ds(9600, 400)])

    stage_waits(0, 0)
    vsrc0, vdst0 = vcopy(0, 0)
    pltpu.async_copy(vsrc0, vdst0, gsem0)
    pltpu.async_copy(
        constraint_hbm.at[cb.at[pl.ds(0, CHUNK)]], rows0, gsem0)
    vsrc1, vdst1 = vcopy(1, 1)
    pltpu.async_copy(vsrc1, vdst1, gsem1)
    pltpu.async_copy(
        constraint_hbm.at[cb.at[pl.ds(CHUNK, CHUNK)]], rows1, gsem1)

    plsc.subcore_barrier()

    def chunk_body(kk, _):
      t = lax.div(kk, SCHUNKS)
      r = lax.rem(kk, SCHUNKS)
      ib = lax.rem(t, 2)

      @pl.when((r == 0) & (t + 1 < N_STAGES))
      def _():
        stage_copies(t + 1, lax.rem(t + 1, 2))

      for b in range(4):
        rbuf, gsem, ssem = gbufs[b]
        nb = (b + 2) % 4
        nbuf, ngsem, nssem = gbufs[nb]

        @pl.when(lax.rem(kk, 4) == b)
        def _(rbuf=rbuf, gsem=gsem, ssem=ssem,
              nbuf=nbuf, ngsem=ngsem, nssem=nssem):
          vsrc, vdst = vcopy(kk, lax.rem(kk, 4))
          pltpu.make_async_copy(vsrc, vdst, gsem).wait()
          coff = pl.multiple_of(ioff(ib) + r * CHUNK, 8)
          pltpu.make_async_copy(
              constraint_hbm.at[cb.at[pl.ds(coff, CHUNK)]],
              rbuf, gsem).wait()

          @pl.when(kk >= 2)
          def _():
            pltpu.make_async_copy(
                nbuf, acc.at[vst.at[0, 0]], nssem).wait()

          @pl.when(kk + 2 < N_CHUNKS)
          def _():
            kk2 = kk + 2
            t2 = lax.div(kk2, SCHUNKS)
            r2 = lax.rem(kk2, SCHUNKS)
            ib2 = lax.rem(t2, 2)

            @pl.when(r2 == 0)
            def _():
              stage_waits(t2, ib2)

            vsrc2, vdst2 = vcopy(kk2, lax.rem(kk2, 4))
            pltpu.async_copy(vsrc2, vdst2, ngsem)
            coff2 = pl.multiple_of(ioff(ib2) + r2 * CHUNK, 8)
            pltpu.async_copy(
                constraint_hbm.at[cb.at[pl.ds(coff2, CHUNK)]],
                nbuf, ngsem)

          def scale_grp(g, _):
            w16 = wb[pl.ds(ioff(ib) + r * CHUNK + g * 16, 16)]
            for l in range(16):
              wsc = w16[l]
              e = g * 16 + l
              for j in range(D // 16):
                rbuf[e, pl.ds(j * 16, 16)] = rbuf[e, pl.ds(j * 16, 16)] * wsc
            return 0

          lax.fori_loop(0, CHUNK // 16, scale_grp, 0)

          pltpu.async_copy(
              rbuf, acc.at[vst.at[lax.rem(kk, 4), 0]], ssem, add=True)

      return 0

    lax.fori_loop(0, N_CHUNKS, chunk_body, 0)

    pltpu.make_async_copy(rows3, acc.at[vst.at[0, 0]], ssem3).wait()
    pltpu.make_async_copy(rows0, acc.at[vst.at[0, 0]], ssem0).wait()
    plsc.subcore_barrier()

    pltpu.sync_copy(cand_hbm.at[s], q2d)
    for q in range(CAND_CHUNKS):
      rbuf, sem, _ = gbufs[q % 2]
      cbase = s * CAND_PER_TILE + q * CHUNK
      if q >= 2:
        pbase = s * CAND_PER_TILE + (q - 2) * CHUNK
        pltpu.make_async_copy(
            rbuf, outp_hbm.at[c, pl.ds(pbase, CHUNK)], sem).wait()
      pltpu.sync_copy(acc.at[q2d.at[q]], rbuf)
      pltpu.async_copy(rbuf, outp_hbm.at[c, pl.ds(cbase, CHUNK)], sem)
    for q in range(CAND_CHUNKS - 2, CAND_CHUNKS):
      rbuf, sem, _ = gbufs[q % 2]
      cbase = s * CAND_PER_TILE + q * CHUNK
      pltpu.make_async_copy(
          rbuf, outp_hbm.at[c, pl.ds(cbase, CHUNK)], sem).wait()

  return k(constraint, cflat, vflat, wflat, cand_pad, zeros)


def _combine(partials):
  def body(p_ref, o_ref):
    o_ref[...] = p_ref[0] + p_ref[1]

  blk = 1000
  return pl.pallas_call(
      body,
      grid=(N_CAND // blk,),
      in_specs=[pl.BlockSpec((NC, blk, D), lambda i: (0, i, 0))],
      out_specs=pl.BlockSpec((blk, D), lambda i: (i, 0)),
      out_shape=jax.ShapeDtypeStruct((N_CAND, D), jnp.float32),
  )(partials)


def kernel(constraint, variable, cv_edge_index, edge_attr, cand_mask):
  cflat = cv_edge_index[0]
  vflat = cv_edge_index[1]
  wflat = edge_attr[:, 0]
  cand_pad = jnp.concatenate(
      [cand_mask, jnp.zeros((CAND_PAD - N_CAND,), jnp.int32)]
  ).reshape(NS, CAND_CHUNKS, CHUNK)
  zeros = jnp.zeros_like(variable)
  partials = _sc_kernel(constraint, cflat, vflat, wflat, cand_pad, zeros)
  return _combine(partials)
